# Initial kernel scaffold; baseline (speedup 1.0000x reference)
import jax
import jax.numpy as jnp
from jax.experimental import pallas as pl


def kernel(x, edge_index, edge_attr, batch, W1, We, W2, Wc1, bc1, Wc2, bc2):
    raise NotImplementedError("write your pallas kernel here")


# speedup vs baseline: 2.3165x; 2.3165x over previous
import functools

import jax
import jax.numpy as jnp
from jax import lax
from jax.experimental import pallas as pl
from jax.experimental.pallas import tpu as pltpu
from jax.experimental.pallas import tpu_sc as plsc

N_NODES_C = 10000
N_EDGES_C = 160000
D_FEAT_C = 256
HIDDEN_C = 256
DH = 128
N_GRAPHS_C = 64
N_CLASSES_C = 10

NSUB = 16
EPB = 80
EDGES_PER_SUB = N_EDGES_C // NSUB
ROWS_PER_SUB = N_NODES_C // NSUB


def _tc_h0(x, W1):
    B = 2000

    def body(x_ref, w_ref, o_ref):
        o_ref[...] = jnp.dot(x_ref[...], w_ref[...],
                             preferred_element_type=jnp.float32)

    return pl.pallas_call(
        body,
        grid=(N_NODES_C // B,),
        in_specs=[
            pl.BlockSpec((B, D_FEAT_C), lambda i: (i, 0)),
            pl.BlockSpec((D_FEAT_C, HIDDEN_C), lambda i: (0, 0)),
        ],
        out_specs=pl.BlockSpec((B, HIDDEN_C), lambda i: (i, 0)),
        out_shape=jax.ShapeDtypeStruct((N_NODES_C, HIDDEN_C), jnp.float32),
    )(x, W1)


def _tc_edge(edge_attr, We):
    B = 4000

    def body(a_ref, w_ref, oa_ref, ob_ref):
        e = jnp.dot(a_ref[...], w_ref[...], preferred_element_type=jnp.float32)
        oa_ref[...] = e[:, :DH]
        ob_ref[...] = e[:, DH:]

    return pl.pallas_call(
        body,
        grid=(N_EDGES_C // B,),
        in_specs=[
            pl.BlockSpec((B, 16), lambda i: (i, 0)),
            pl.BlockSpec((16, HIDDEN_C), lambda i: (0, 0)),
        ],
        out_specs=[
            pl.BlockSpec((B, DH), lambda i: (i, 0)),
            pl.BlockSpec((B, DH), lambda i: (i, 0)),
        ],
        out_shape=[
            jax.ShapeDtypeStruct((N_EDGES_C, DH), jnp.float32),
            jax.ShapeDtypeStruct((N_EDGES_C, DH), jnp.float32),
        ],
    )(edge_attr, We)


def _sc_aggregate(h02, Ea, Eb, src, dst, zrows):
    mesh = plsc.VectorSubcoreMesh(core_axis_name="c", subcore_axis_name="s")

    @functools.partial(
        pl.kernel,
        out_type=[
            jax.ShapeDtypeStruct((N_NODES_C, DH), jnp.float32),
            jax.ShapeDtypeStruct((N_NODES_C, DH), jnp.float32),
        ],
        mesh=mesh,
        scratch_types=[
            pltpu.VMEM((EPB,), jnp.int32),
            pltpu.VMEM((EPB,), jnp.int32),
            pltpu.VMEM((EPB,), jnp.int32),
            pltpu.VMEM((EPB, DH), jnp.float32),
            pltpu.VMEM((EPB, DH), jnp.float32),
            pltpu.VMEM_SHARED((N_NODES_C, DH), jnp.float32),
            pltpu.SemaphoreType.DMA,
            pltpu.SemaphoreType.DMA,
        ],
    )
    def sc_kernel(h02_hbm, ea_hbm, eb_hbm, src_hbm, dst_hbm, z_hbm,
                  oa_hbm, ob_hbm,
                  didx, sidx, sidx2, gbuf, ebuf, agg_sh, sem_g, sem_e):
        c = lax.axis_index("c")
        s = lax.axis_index("s")
        ebase = s * EDGES_PER_SUB
        RCH = 40

        @pl.loop(s * RCH, N_NODES_C, step=NSUB * RCH)
        def _(r0):
            pltpu.sync_copy(z_hbm.at[pl.ds(r0, RCH)],
                            agg_sh.at[pl.ds(r0, RCH)])
        plsc.subcore_barrier()

        def edge_loop(e_hbm, cval):
            @pl.loop(0, EDGES_PER_SUB, step=EPB)
            def _(eo):
                e0 = ebase + eo
                pltpu.sync_copy(dst_hbm.at[pl.ds(e0, EPB)], didx)
                pltpu.sync_copy(src_hbm.at[pl.ds(e0, EPB)], sidx)
                for kk in range(EPB // 16):
                    v = sidx[pl.ds(kk * 16, 16)]
                    sidx2[pl.ds(kk * 16, 16)] = v + v + cval
                cp_g = pltpu.async_copy(h02_hbm.at[sidx2], gbuf, sem_g)
                cp_e = pltpu.async_copy(e_hbm.at[pl.ds(e0, EPB)], ebuf, sem_e)
                cp_g.wait()
                cp_e.wait()

                @pl.loop(0, EPB)
                def _(i):
                    for j in range(DH // 16):
                        sl = pl.ds(j * 16, 16)
                        gbuf[i, sl] = jnp.maximum(
                            gbuf[i, sl] + ebuf[i, sl], 0.0)

                pltpu.sync_copy(gbuf, agg_sh.at[didx], add=True)

        @pl.when(c == 0)
        def _():
            edge_loop(ea_hbm, 0)

        @pl.when(c == 1)
        def _():
            edge_loop(eb_hbm, 1)

        plsc.subcore_barrier()

        @pl.when(c == 0)
        def _():
            @pl.loop(s * RCH, N_NODES_C, step=NSUB * RCH)
            def _(r0):
                pltpu.sync_copy(agg_sh.at[pl.ds(r0, RCH)],
                                oa_hbm.at[pl.ds(r0, RCH)])

        @pl.when(c == 1)
        def _():
            @pl.loop(s * RCH, N_NODES_C, step=NSUB * RCH)
            def _(r0):
                pltpu.sync_copy(agg_sh.at[pl.ds(r0, RCH)],
                                ob_hbm.at[pl.ds(r0, RCH)])

    return sc_kernel(h02, Ea, Eb, src, dst, zrows)


def _tc_hidden_pool(h0, agga, aggb, W2, batch3):
    B = 1000

    def body(h0_ref, aa_ref, ab_ref, w2_ref, b_ref, sums_ref, cnts_ref):
        i = pl.program_id(0)
        agg = jnp.concatenate([aa_ref[...], ab_ref[...]], axis=1)
        h = jnp.maximum(
            h0_ref[...] + jnp.dot(agg, w2_ref[...],
                                  preferred_element_type=jnp.float32), 0.0)
        b = b_ref[...].reshape(B)
        onehot = (b[:, None] == lax.broadcasted_iota(
            jnp.int32, (B, N_GRAPHS_C), 1)).astype(jnp.float32)
        ps = lax.dot_general(onehot, h, (((0,), (0,)), ((), ())),
                             preferred_element_type=jnp.float32)
        pc = jnp.sum(onehot, axis=0, keepdims=True)

        @pl.when(i == 0)
        def _():
            sums_ref[...] = jnp.zeros_like(sums_ref)
            cnts_ref[...] = jnp.zeros_like(cnts_ref)

        sums_ref[...] += ps
        cnts_ref[...] += pc

    return pl.pallas_call(
        body,
        grid=(N_NODES_C // B,),
        in_specs=[
            pl.BlockSpec((B, HIDDEN_C), lambda i: (i, 0)),
            pl.BlockSpec((B, DH), lambda i: (i, 0)),
            pl.BlockSpec((B, DH), lambda i: (i, 0)),
            pl.BlockSpec((HIDDEN_C, HIDDEN_C), lambda i: (0, 0)),
            pl.BlockSpec((1, 1, B), lambda i: (i, 0, 0)),
        ],
        out_specs=[
            pl.BlockSpec((N_GRAPHS_C, HIDDEN_C), lambda i: (0, 0)),
            pl.BlockSpec((1, N_GRAPHS_C), lambda i: (0, 0)),
        ],
        out_shape=[
            jax.ShapeDtypeStruct((N_GRAPHS_C, HIDDEN_C), jnp.float32),
            jax.ShapeDtypeStruct((1, N_GRAPHS_C), jnp.float32),
        ],
    )(h0, agga, aggb, W2, batch3)


def _tc_head(sums, cnts, Wc1, bc1, Wc2, bc2):
    def body(s_ref, c_ref, w1_ref, b1_ref, w2_ref, b2_ref, o_ref):
        cnt = jnp.maximum(c_ref[...].reshape(N_GRAPHS_C), 1.0)
        g = s_ref[...] / cnt[:, None]
        z = jnp.maximum(
            jnp.dot(g, w1_ref[...], preferred_element_type=jnp.float32)
            + b1_ref[...], 0.0)
        o_ref[...] = (jnp.dot(z, w2_ref[...],
                              preferred_element_type=jnp.float32)
                      + b2_ref[...])

    return pl.pallas_call(
        body,
        out_shape=jax.ShapeDtypeStruct((N_GRAPHS_C, N_CLASSES_C), jnp.float32),
    )(sums, cnts, Wc1, bc1.reshape(1, -1), Wc2, bc2.reshape(1, -1))


def kernel(x, edge_index, edge_attr, batch, W1, We, W2, Wc1, bc1, Wc2, bc2):
    src = edge_index[0]
    dst = edge_index[1]
    h0 = _tc_h0(x, W1)
    Ea, Eb = _tc_edge(edge_attr, We)
    zrows = jnp.zeros((N_NODES_C, DH), dtype=jnp.float32)
    agga, aggb = _sc_aggregate(h0.reshape(2 * N_NODES_C, DH), Ea, Eb,
                               src, dst, zrows)
    batch3 = batch.reshape(N_NODES_C // 1000, 1, 1000)
    sums, cnts = _tc_hidden_pool(h0, agga, aggb, W2, batch3)
    return _tc_head(sums, cnts, Wc1, bc1, Wc2, bc2)


# speedup vs baseline: 3.5935x; 1.5513x over previous
import functools

import jax
import jax.numpy as jnp
from jax import lax
from jax.experimental import pallas as pl
from jax.experimental.pallas import tpu as pltpu
from jax.experimental.pallas import tpu_sc as plsc

N_NODES_C = 10000
N_EDGES_C = 160000
D_FEAT_C = 256
HIDDEN_C = 256
DH = 128
N_GRAPHS_C = 64
N_CLASSES_C = 10

NSUB = 16
EPB = 40
EDGES_PER_SUB = N_EDGES_C // NSUB
ROWS_PER_SUB = N_NODES_C // NSUB
ROUND = 2000
NBLK_R = ROUND // EPB
NROUND = EDGES_PER_SUB // ROUND


def _tc_h0(x, W1):
    B = 2000

    def body(x_ref, w_ref, o_ref):
        o_ref[...] = jnp.dot(x_ref[...], w_ref[...],
                             preferred_element_type=jnp.float32)

    return pl.pallas_call(
        body,
        grid=(N_NODES_C // B,),
        in_specs=[
            pl.BlockSpec((B, D_FEAT_C), lambda i: (i, 0)),
            pl.BlockSpec((D_FEAT_C, HIDDEN_C), lambda i: (0, 0)),
        ],
        out_specs=pl.BlockSpec((B, HIDDEN_C), lambda i: (i, 0)),
        out_shape=jax.ShapeDtypeStruct((N_NODES_C, HIDDEN_C), jnp.float32),
    )(x, W1)


def _tc_edge(edge_attr, We):
    B = 4000

    def body(a_ref, w_ref, oa_ref, ob_ref):
        e = jnp.dot(a_ref[...], w_ref[...], preferred_element_type=jnp.float32)
        oa_ref[...] = e[:, :DH]
        ob_ref[...] = e[:, DH:]

    return pl.pallas_call(
        body,
        grid=(N_EDGES_C // B,),
        in_specs=[
            pl.BlockSpec((B, 16), lambda i: (i, 0)),
            pl.BlockSpec((16, HIDDEN_C), lambda i: (0, 0)),
        ],
        out_specs=[
            pl.BlockSpec((B, DH), lambda i: (i, 0)),
            pl.BlockSpec((B, DH), lambda i: (i, 0)),
        ],
        out_shape=[
            jax.ShapeDtypeStruct((N_EDGES_C, DH), jnp.float32),
            jax.ShapeDtypeStruct((N_EDGES_C, DH), jnp.float32),
        ],
    )(edge_attr, We)


def _sc_aggregate(h02, Ea, Eb, src, dst3, zrows):
    mesh = plsc.VectorSubcoreMesh(core_axis_name="c", subcore_axis_name="s")

    @functools.partial(
        pl.kernel,
        out_type=[
            jax.ShapeDtypeStruct((N_NODES_C, DH), jnp.float32),
            jax.ShapeDtypeStruct((N_NODES_C, DH), jnp.float32),
        ],
        mesh=mesh,
        scratch_types=[
            pltpu.VMEM((ROUND,), jnp.int32),
            pltpu.VMEM((NBLK_R, EPB), jnp.int32),
            pltpu.VMEM((EPB, DH), jnp.float32),
            pltpu.VMEM((EPB, DH), jnp.float32),
            pltpu.VMEM((EPB, DH), jnp.float32),
            pltpu.VMEM((EPB, DH), jnp.float32),
            pltpu.VMEM((EPB, DH), jnp.float32),
            pltpu.VMEM((EPB, DH), jnp.float32),
            pltpu.VMEM_SHARED((N_NODES_C, DH), jnp.float32),
            pltpu.SemaphoreType.DMA,
            pltpu.SemaphoreType.DMA,
            pltpu.SemaphoreType.DMA,
            pltpu.SemaphoreType.DMA,
            pltpu.SemaphoreType.DMA,
            pltpu.SemaphoreType.DMA,
        ],
    )
    def sc_kernel(h02_hbm, ea_hbm, eb_hbm, src_hbm, dst4_hbm, z_hbm,
                  oa_hbm, ob_hbm,
                  sidx_r, didx_r, gb0, gb1, eb0, eb1, sb0, sb1,
                  agg_sh, sg0, sg1, se0, se1, ss0, ss1):
        c = lax.axis_index("c")
        s = lax.axis_index("s")
        ebase = s * EDGES_PER_SUB
        RCH = 40

        @pl.loop(s * RCH, N_NODES_C, step=NSUB * RCH)
        def _(r0):
            pltpu.sync_copy(z_hbm.at[pl.ds(r0, RCH)],
                            agg_sh.at[pl.ds(r0, RCH)])
        plsc.subcore_barrier()

        def edge_loop(e_hbm, cval):
            @pl.loop(0, NROUND)
            def _(r):
                rb = ebase + r * ROUND

                pltpu.sync_copy(src_hbm.at[pl.ds(rb, ROUND)], sidx_r)
                pltpu.sync_copy(dst4_hbm.at[s, r], didx_r)

                @pl.loop(0, ROUND, step=16)
                def _(i):
                    v = sidx_r[pl.ds(i, 16)]
                    sidx_r[pl.ds(i, 16)] = v + v + cval

                def fire(gb, eb, sg, se, blk):
                    pltpu.async_copy(
                        h02_hbm.at[sidx_r.at[pl.ds(blk * EPB, EPB)]], gb, sg)
                    pltpu.async_copy(
                        e_hbm.at[pl.ds(rb + blk * EPB, EPB)], eb, se)

                def wait_in(gb, eb, sg, se, blk):
                    pltpu.make_async_copy(
                        h02_hbm.at[sidx_r.at[pl.ds(blk * EPB, EPB)]],
                        gb, sg).wait()
                    pltpu.make_async_copy(
                        e_hbm.at[pl.ds(rb + blk * EPB, EPB)], eb, se).wait()

                def wait_scat(sb, ss, blk):
                    pltpu.make_async_copy(
                        sb, agg_sh.at[didx_r.at[blk]], ss).wait()

                def compute(gb, eb, sb):
                    @pl.loop(0, EPB)
                    def _(i):
                        for j in range(DH // 16):
                            sl = pl.ds(j * 16, 16)
                            sb[i, sl] = jnp.maximum(
                                gb[i, sl] + eb[i, sl], 0.0)

                def drain(gb, eb, sb, sg, se, ss, blk, refire):
                    wait_in(gb, eb, sg, se, blk)

                    @pl.when(blk >= 2)
                    def _():
                        wait_scat(sb, ss, blk)

                    compute(gb, eb, sb)
                    pltpu.async_copy(sb, agg_sh.at[didx_r.at[blk]], ss,
                                     add=True)
                    if refire:
                        fire(gb, eb, sg, se, blk + 2)

                fire(gb0, eb0, sg0, se0, 0)
                fire(gb1, eb1, sg1, se1, 1)

                @pl.loop(0, (NBLK_R - 2) // 2)
                def _(p):
                    drain(gb0, eb0, sb0, sg0, se0, ss0, 2 * p, True)
                    drain(gb1, eb1, sb1, sg1, se1, ss1, 2 * p + 1, True)

                drain(gb0, eb0, sb0, sg0, se0, ss0, NBLK_R - 2, False)
                drain(gb1, eb1, sb1, sg1, se1, ss1, NBLK_R - 1, False)
                wait_scat(sb0, ss0, NBLK_R - 2)
                wait_scat(sb1, ss1, NBLK_R - 1)

        @pl.when(c == 0)
        def _():
            edge_loop(ea_hbm, 0)

        @pl.when(c == 1)
        def _():
            edge_loop(eb_hbm, 1)

        plsc.subcore_barrier()

        @pl.when(c == 0)
        def _():
            @pl.loop(s * RCH, N_NODES_C, step=NSUB * RCH)
            def _(r0):
                pltpu.sync_copy(agg_sh.at[pl.ds(r0, RCH)],
                                oa_hbm.at[pl.ds(r0, RCH)])

        @pl.when(c == 1)
        def _():
            @pl.loop(s * RCH, N_NODES_C, step=NSUB * RCH)
            def _(r0):
                pltpu.sync_copy(agg_sh.at[pl.ds(r0, RCH)],
                                ob_hbm.at[pl.ds(r0, RCH)])

    return sc_kernel(h02, Ea, Eb, src, dst3, zrows)


def _tc_hidden_pool(h0, agga, aggb, W2, batch3):
    B = 1000

    def body(h0_ref, aa_ref, ab_ref, w2_ref, b_ref, sums_ref, cnts_ref):
        i = pl.program_id(0)
        agg = jnp.concatenate([aa_ref[...], ab_ref[...]], axis=1)
        h = jnp.maximum(
            h0_ref[...] + jnp.dot(agg, w2_ref[...],
                                  preferred_element_type=jnp.float32), 0.0)
        b = b_ref[...].reshape(B)
        onehot = (b[:, None] == lax.broadcasted_iota(
            jnp.int32, (B, N_GRAPHS_C), 1)).astype(jnp.float32)
        ps = lax.dot_general(onehot, h, (((0,), (0,)), ((), ())),
                             preferred_element_type=jnp.float32)
        pc = jnp.sum(onehot, axis=0, keepdims=True)

        @pl.when(i == 0)
        def _():
            sums_ref[...] = jnp.zeros_like(sums_ref)
            cnts_ref[...] = jnp.zeros_like(cnts_ref)

        sums_ref[...] += ps
        cnts_ref[...] += pc

    return pl.pallas_call(
        body,
        grid=(N_NODES_C // B,),
        in_specs=[
            pl.BlockSpec((B, HIDDEN_C), lambda i: (i, 0)),
            pl.BlockSpec((B, DH), lambda i: (i, 0)),
            pl.BlockSpec((B, DH), lambda i: (i, 0)),
            pl.BlockSpec((HIDDEN_C, HIDDEN_C), lambda i: (0, 0)),
            pl.BlockSpec((1, 1, B), lambda i: (i, 0, 0)),
        ],
        out_specs=[
            pl.BlockSpec((N_GRAPHS_C, HIDDEN_C), lambda i: (0, 0)),
            pl.BlockSpec((1, N_GRAPHS_C), lambda i: (0, 0)),
        ],
        out_shape=[
            jax.ShapeDtypeStruct((N_GRAPHS_C, HIDDEN_C), jnp.float32),
            jax.ShapeDtypeStruct((1, N_GRAPHS_C), jnp.float32),
        ],
    )(h0, agga, aggb, W2, batch3)


def _tc_head(sums, cnts, Wc1, bc1, Wc2, bc2):
    def body(s_ref, c_ref, w1_ref, b1_ref, w2_ref, b2_ref, o_ref):
        cnt = jnp.maximum(c_ref[...].reshape(N_GRAPHS_C), 1.0)
        g = s_ref[...] / cnt[:, None]
        z = jnp.maximum(
            jnp.dot(g, w1_ref[...], preferred_element_type=jnp.float32)
            + b1_ref[...], 0.0)
        o_ref[...] = (jnp.dot(z, w2_ref[...],
                              preferred_element_type=jnp.float32)
                      + b2_ref[...])

    return pl.pallas_call(
        body,
        out_shape=jax.ShapeDtypeStruct((N_GRAPHS_C, N_CLASSES_C), jnp.float32),
    )(sums, cnts, Wc1, bc1.reshape(1, -1), Wc2, bc2.reshape(1, -1))


def kernel(x, edge_index, edge_attr, batch, W1, We, W2, Wc1, bc1, Wc2, bc2):
    src = edge_index[0]
    dst = edge_index[1]
    h0 = _tc_h0(x, W1)
    Ea, Eb = _tc_edge(edge_attr, We)
    zrows = jnp.zeros((N_NODES_C, DH), dtype=jnp.float32)
    dst4 = dst.reshape(NSUB, NROUND, NBLK_R, EPB)
    agga, aggb = _sc_aggregate(h0.reshape(2 * N_NODES_C, DH), Ea, Eb,
                               src, dst4, zrows)
    batch3 = batch.reshape(N_NODES_C // 1000, 1, 1000)
    sums, cnts = _tc_hidden_pool(h0, agga, aggb, W2, batch3)
    return _tc_head(sums, cnts, Wc1, bc1, Wc2, bc2)


# speedup vs baseline: 3.7195x; 1.0351x over previous
import functools

import jax
import jax.numpy as jnp
from jax import lax
from jax.experimental import pallas as pl
from jax.experimental.pallas import tpu as pltpu
from jax.experimental.pallas import tpu_sc as plsc

N_NODES_C = 10000
N_EDGES_C = 160000
D_FEAT_C = 256
HIDDEN_C = 256
DH = 128
N_GRAPHS_C = 64
N_CLASSES_C = 10

NSUB = 16
EPB = 40
EDGES_PER_SUB = N_EDGES_C // NSUB
ROWS_PER_SUB = N_NODES_C // NSUB
ROUND = 2000
NBLK_R = ROUND // EPB
NROUND = EDGES_PER_SUB // ROUND


def _tc_h0(x, W1):
    B = 2000

    def body(x_ref, w_ref, oa_ref, ob_ref):
        h = jnp.dot(x_ref[...], w_ref[...], preferred_element_type=jnp.float32)
        oa_ref[...] = h[:, :DH]
        ob_ref[...] = h[:, DH:]

    return pl.pallas_call(
        body,
        grid=(N_NODES_C // B,),
        in_specs=[
            pl.BlockSpec((B, D_FEAT_C), lambda i: (i, 0)),
            pl.BlockSpec((D_FEAT_C, HIDDEN_C), lambda i: (0, 0)),
        ],
        out_specs=[
            pl.BlockSpec((B, DH), lambda i: (i, 0)),
            pl.BlockSpec((B, DH), lambda i: (i, 0)),
        ],
        out_shape=[
            jax.ShapeDtypeStruct((N_NODES_C, DH), jnp.float32),
            jax.ShapeDtypeStruct((N_NODES_C, DH), jnp.float32),
        ],
    )(x, W1)


def _tc_edge(edge_attr, We):
    B = 4000

    def body(a_ref, w_ref, oa_ref, ob_ref):
        e = jnp.dot(a_ref[...], w_ref[...], preferred_element_type=jnp.float32)
        oa_ref[...] = e[:, :DH]
        ob_ref[...] = e[:, DH:]

    return pl.pallas_call(
        body,
        grid=(N_EDGES_C // B,),
        in_specs=[
            pl.BlockSpec((B, 16), lambda i: (i, 0)),
            pl.BlockSpec((16, HIDDEN_C), lambda i: (0, 0)),
        ],
        out_specs=[
            pl.BlockSpec((B, DH), lambda i: (i, 0)),
            pl.BlockSpec((B, DH), lambda i: (i, 0)),
        ],
        out_shape=[
            jax.ShapeDtypeStruct((N_EDGES_C, DH), jnp.float32),
            jax.ShapeDtypeStruct((N_EDGES_C, DH), jnp.float32),
        ],
    )(edge_attr, We)


def _sc_aggregate(h0a, h0b, Ea, Eb, src, dst4, zrows):
    mesh = plsc.VectorSubcoreMesh(core_axis_name="c", subcore_axis_name="s")

    @functools.partial(
        pl.kernel,
        out_type=[
            jax.ShapeDtypeStruct((N_NODES_C, DH), jnp.float32),
            jax.ShapeDtypeStruct((N_NODES_C, DH), jnp.float32),
        ],
        mesh=mesh,
        scratch_types=[
            pltpu.VMEM((ROUND,), jnp.int32),
            pltpu.VMEM((NBLK_R, EPB), jnp.int32),
            pltpu.VMEM((EPB, DH), jnp.float32),
            pltpu.VMEM((EPB, DH), jnp.float32),
            pltpu.VMEM((EPB, DH), jnp.float32),
            pltpu.VMEM((EPB, DH), jnp.float32),
            pltpu.VMEM((EPB, DH), jnp.float32),
            pltpu.VMEM((EPB, DH), jnp.float32),
            pltpu.VMEM_SHARED((N_NODES_C, DH), jnp.float32),
            pltpu.SemaphoreType.DMA,
            pltpu.SemaphoreType.DMA,
            pltpu.SemaphoreType.DMA,
            pltpu.SemaphoreType.DMA,
            pltpu.SemaphoreType.DMA,
            pltpu.SemaphoreType.DMA,
        ],
    )
    def sc_kernel(ha_hbm, hb_hbm, ea_hbm, eb_hbm, src_hbm, dst4_hbm, z_hbm,
                  oa_hbm, ob_hbm,
                  sidx_r, didx_r, gb0, gb1, eb0, eb1, sb0, sb1,
                  agg_sh, sg0, sg1, se0, se1, ss0, ss1):
        c = lax.axis_index("c")
        s = lax.axis_index("s")
        ebase = s * EDGES_PER_SUB
        RCH = 40

        @pl.loop(s * RCH, N_NODES_C, step=NSUB * RCH)
        def _(r0):
            pltpu.sync_copy(z_hbm.at[pl.ds(r0, RCH)],
                            agg_sh.at[pl.ds(r0, RCH)])
        plsc.subcore_barrier()

        def edge_loop(h_hbm, e_hbm):
            @pl.loop(0, NROUND)
            def _(r):
                rb = ebase + r * ROUND

                pltpu.sync_copy(src_hbm.at[pl.ds(rb, ROUND)], sidx_r)
                pltpu.sync_copy(dst4_hbm.at[s, r], didx_r)

                def fire(gb, eb, sg, se, blk):
                    pltpu.async_copy(
                        h_hbm.at[sidx_r.at[pl.ds(blk * EPB, EPB)]], gb, sg)
                    pltpu.async_copy(
                        e_hbm.at[pl.ds(rb + blk * EPB, EPB)], eb, se)

                def wait_in(gb, eb, sg, se, blk):
                    pltpu.make_async_copy(
                        h_hbm.at[sidx_r.at[pl.ds(blk * EPB, EPB)]],
                        gb, sg).wait()
                    pltpu.make_async_copy(
                        e_hbm.at[pl.ds(rb + blk * EPB, EPB)], eb, se).wait()

                def wait_scat(sb, ss, blk):
                    pltpu.make_async_copy(
                        sb, agg_sh.at[didx_r.at[blk]], ss).wait()

                def compute(gb, eb, sb):
                    @pl.loop(0, EPB)
                    def _(i):
                        for j in range(DH // 16):
                            sl = pl.ds(j * 16, 16)
                            sb[i, sl] = jnp.maximum(
                                gb[i, sl] + eb[i, sl], 0.0)

                def drain(gb, eb, sb, sg, se, ss, blk, refire):
                    wait_in(gb, eb, sg, se, blk)

                    @pl.when(blk >= 2)
                    def _():
                        wait_scat(sb, ss, blk)

                    compute(gb, eb, sb)
                    pltpu.async_copy(sb, agg_sh.at[didx_r.at[blk]], ss,
                                     add=True)
                    if refire:
                        fire(gb, eb, sg, se, blk + 2)

                fire(gb0, eb0, sg0, se0, 0)
                fire(gb1, eb1, sg1, se1, 1)

                @pl.loop(0, (NBLK_R - 2) // 2)
                def _(p):
                    drain(gb0, eb0, sb0, sg0, se0, ss0, 2 * p, True)
                    drain(gb1, eb1, sb1, sg1, se1, ss1, 2 * p + 1, True)

                drain(gb0, eb0, sb0, sg0, se0, ss0, NBLK_R - 2, False)
                drain(gb1, eb1, sb1, sg1, se1, ss1, NBLK_R - 1, False)
                wait_scat(sb0, ss0, NBLK_R - 2)
                wait_scat(sb1, ss1, NBLK_R - 1)

        @pl.when(c == 0)
        def _():
            edge_loop(ha_hbm, ea_hbm)

        @pl.when(c == 1)
        def _():
            edge_loop(hb_hbm, eb_hbm)

        plsc.subcore_barrier()

        @pl.when(c == 0)
        def _():
            @pl.loop(s * RCH, N_NODES_C, step=NSUB * RCH)
            def _(r0):
                pltpu.sync_copy(agg_sh.at[pl.ds(r0, RCH)],
                                oa_hbm.at[pl.ds(r0, RCH)])

        @pl.when(c == 1)
        def _():
            @pl.loop(s * RCH, N_NODES_C, step=NSUB * RCH)
            def _(r0):
                pltpu.sync_copy(agg_sh.at[pl.ds(r0, RCH)],
                                ob_hbm.at[pl.ds(r0, RCH)])

    return sc_kernel(h0a, h0b, Ea, Eb, src, dst4, zrows)


def _tc_hidden_pool(h0a, h0b, agga, aggb, W2, batch3):
    B = 1000

    def body(ha_ref, hb_ref, aa_ref, ab_ref, w2_ref, b_ref,
             sums_ref, cnts_ref):
        i = pl.program_id(0)
        h0 = jnp.concatenate([ha_ref[...], hb_ref[...]], axis=1)
        agg = jnp.concatenate([aa_ref[...], ab_ref[...]], axis=1)
        h = jnp.maximum(
            h0 + jnp.dot(agg, w2_ref[...],
                         preferred_element_type=jnp.float32), 0.0)
        b = b_ref[...].reshape(B)
        onehot = (b[:, None] == lax.broadcasted_iota(
            jnp.int32, (B, N_GRAPHS_C), 1)).astype(jnp.float32)
        ps = lax.dot_general(onehot, h, (((0,), (0,)), ((), ())),
                             preferred_element_type=jnp.float32)
        pc = jnp.sum(onehot, axis=0, keepdims=True)

        @pl.when(i == 0)
        def _():
            sums_ref[...] = jnp.zeros_like(sums_ref)
            cnts_ref[...] = jnp.zeros_like(cnts_ref)

        sums_ref[...] += ps
        cnts_ref[...] += pc

    return pl.pallas_call(
        body,
        grid=(N_NODES_C // B,),
        in_specs=[
            pl.BlockSpec((B, DH), lambda i: (i, 0)),
            pl.BlockSpec((B, DH), lambda i: (i, 0)),
            pl.BlockSpec((B, DH), lambda i: (i, 0)),
            pl.BlockSpec((B, DH), lambda i: (i, 0)),
            pl.BlockSpec((HIDDEN_C, HIDDEN_C), lambda i: (0, 0)),
            pl.BlockSpec((1, 1, B), lambda i: (i, 0, 0)),
        ],
        out_specs=[
            pl.BlockSpec((N_GRAPHS_C, HIDDEN_C), lambda i: (0, 0)),
            pl.BlockSpec((1, N_GRAPHS_C), lambda i: (0, 0)),
        ],
        out_shape=[
            jax.ShapeDtypeStruct((N_GRAPHS_C, HIDDEN_C), jnp.float32),
            jax.ShapeDtypeStruct((1, N_GRAPHS_C), jnp.float32),
        ],
    )(h0a, h0b, agga, aggb, W2, batch3)


def _tc_head(sums, cnts, Wc1, bc1, Wc2, bc2):
    def body(s_ref, c_ref, w1_ref, b1_ref, w2_ref, b2_ref, o_ref):
        cnt = jnp.maximum(c_ref[...].reshape(N_GRAPHS_C), 1.0)
        g = s_ref[...] / cnt[:, None]
        z = jnp.maximum(
            jnp.dot(g, w1_ref[...], preferred_element_type=jnp.float32)
            + b1_ref[...], 0.0)
        o_ref[...] = (jnp.dot(z, w2_ref[...],
                              preferred_element_type=jnp.float32)
                      + b2_ref[...])

    return pl.pallas_call(
        body,
        out_shape=jax.ShapeDtypeStruct((N_GRAPHS_C, N_CLASSES_C), jnp.float32),
    )(sums, cnts, Wc1, bc1.reshape(1, -1), Wc2, bc2.reshape(1, -1))


def kernel(x, edge_index, edge_attr, batch, W1, We, W2, Wc1, bc1, Wc2, bc2):
    src = edge_index[0]
    dst = edge_index[1]
    h0a, h0b = _tc_h0(x, W1)
    Ea, Eb = _tc_edge(edge_attr, We)
    zrows = jnp.zeros((N_NODES_C, DH), dtype=jnp.float32)
    dst4 = dst.reshape(NSUB, NROUND, NBLK_R, EPB)
    agga, aggb = _sc_aggregate(h0a, h0b, Ea, Eb, src, dst4, zrows)
    batch3 = batch.reshape(N_NODES_C // 1000, 1, 1000)
    sums, cnts = _tc_hidden_pool(h0a, h0b, agga, aggb, W2, batch3)
    return _tc_head(sums, cnts, Wc1, bc1, Wc2, bc2)


# speedup vs baseline: 3.7753x; 1.0150x over previous
import functools

import jax
import jax.numpy as jnp
from jax import lax
from jax.experimental import pallas as pl
from jax.experimental.pallas import tpu as pltpu
from jax.experimental.pallas import tpu_sc as plsc

N_NODES_C = 10000
N_EDGES_C = 160000
D_FEAT_C = 256
HIDDEN_C = 256
DH = 128
N_GRAPHS_C = 64
N_CLASSES_C = 10

NSUB = 16
EPB = 40
ROUND = 2000
NBLK_R = ROUND // EPB
CHUNK1 = 64000
CHUNK2 = 96000


def _tc_h0(x, W1):
    B = 2000

    def body(x_ref, w_ref, oa_ref, ob_ref):
        h = jnp.dot(x_ref[...], w_ref[...], preferred_element_type=jnp.float32)
        oa_ref[...] = h[:, :DH]
        ob_ref[...] = h[:, DH:]

    return pl.pallas_call(
        body,
        grid=(N_NODES_C // B,),
        in_specs=[
            pl.BlockSpec((B, D_FEAT_C), lambda i: (i, 0)),
            pl.BlockSpec((D_FEAT_C, HIDDEN_C), lambda i: (0, 0)),
        ],
        out_specs=[
            pl.BlockSpec((B, DH), lambda i: (i, 0)),
            pl.BlockSpec((B, DH), lambda i: (i, 0)),
        ],
        out_shape=[
            jax.ShapeDtypeStruct((N_NODES_C, DH), jnp.float32),
            jax.ShapeDtypeStruct((N_NODES_C, DH), jnp.float32),
        ],
    )(x, W1)


def _tc_edge(edge_attr, We):
    B = 4000
    n = edge_attr.shape[0]

    def body(a_ref, w_ref, oa_ref, ob_ref):
        e = jnp.dot(a_ref[...], w_ref[...], preferred_element_type=jnp.float32)
        oa_ref[...] = e[:, :DH]
        ob_ref[...] = e[:, DH:]

    return pl.pallas_call(
        body,
        grid=(n // B,),
        in_specs=[
            pl.BlockSpec((B, 16), lambda i: (i, 0)),
            pl.BlockSpec((16, HIDDEN_C), lambda i: (0, 0)),
        ],
        out_specs=[
            pl.BlockSpec((B, DH), lambda i: (i, 0)),
            pl.BlockSpec((B, DH), lambda i: (i, 0)),
        ],
        out_shape=[
            jax.ShapeDtypeStruct((n, DH), jnp.float32),
            jax.ShapeDtypeStruct((n, DH), jnp.float32),
        ],
    )(edge_attr, We)


def _sc_aggregate(h0a, h0b, Ea, Eb, src, dst4, inita, initb, nround):
    mesh = plsc.VectorSubcoreMesh(core_axis_name="c", subcore_axis_name="s")
    edges_per_sub = nround * ROUND

    @functools.partial(
        pl.kernel,
        out_type=[
            jax.ShapeDtypeStruct((N_NODES_C, DH), jnp.float32),
            jax.ShapeDtypeStruct((N_NODES_C, DH), jnp.float32),
        ],
        mesh=mesh,
        scratch_types=[
            pltpu.VMEM((ROUND,), jnp.int32),
            pltpu.VMEM((NBLK_R, EPB), jnp.int32),
            pltpu.VMEM((EPB, DH), jnp.float32),
            pltpu.VMEM((EPB, DH), jnp.float32),
            pltpu.VMEM((EPB, DH), jnp.float32),
            pltpu.VMEM((EPB, DH), jnp.float32),
            pltpu.VMEM((EPB, DH), jnp.float32),
            pltpu.VMEM((EPB, DH), jnp.float32),
            pltpu.VMEM_SHARED((N_NODES_C, DH), jnp.float32),
            pltpu.SemaphoreType.DMA,
            pltpu.SemaphoreType.DMA,
            pltpu.SemaphoreType.DMA,
            pltpu.SemaphoreType.DMA,
            pltpu.SemaphoreType.DMA,
            pltpu.SemaphoreType.DMA,
        ],
    )
    def sc_kernel(ha_hbm, hb_hbm, ea_hbm, eb_hbm, src_hbm, dst4_hbm,
                  ia_hbm, ib_hbm, oa_hbm, ob_hbm,
                  sidx_r, didx_r, gb0, gb1, eb0, eb1, sb0, sb1,
                  agg_sh, sg0, sg1, se0, se1, ss0, ss1):
        c = lax.axis_index("c")
        s = lax.axis_index("s")
        ebase = s * edges_per_sub
        RCH = 40

        def init_loop(i_hbm):
            @pl.loop(s * RCH, N_NODES_C, step=NSUB * RCH)
            def _(r0):
                pltpu.sync_copy(i_hbm.at[pl.ds(r0, RCH)],
                                agg_sh.at[pl.ds(r0, RCH)])

        @pl.when(c == 0)
        def _():
            init_loop(ia_hbm)

        @pl.when(c == 1)
        def _():
            init_loop(ib_hbm)

        plsc.subcore_barrier()

        def edge_loop(h_hbm, e_hbm):
            @pl.loop(0, nround)
            def _(r):
                rb = ebase + r * ROUND

                pltpu.sync_copy(src_hbm.at[pl.ds(rb, ROUND)], sidx_r)
                pltpu.sync_copy(dst4_hbm.at[s, r], didx_r)

                def fire(gb, eb, sg, se, blk):
                    pltpu.async_copy(
                        h_hbm.at[sidx_r.at[pl.ds(blk * EPB, EPB)]], gb, sg)
                    pltpu.async_copy(
                        e_hbm.at[pl.ds(rb + blk * EPB, EPB)], eb, se)

                def wait_in(gb, eb, sg, se, blk):
                    pltpu.make_async_copy(
                        h_hbm.at[sidx_r.at[pl.ds(blk * EPB, EPB)]],
                        gb, sg).wait()
                    pltpu.make_async_copy(
                        e_hbm.at[pl.ds(rb + blk * EPB, EPB)], eb, se).wait()

                def wait_scat(sb, ss, blk):
                    pltpu.make_async_copy(
                        sb, agg_sh.at[didx_r.at[blk]], ss).wait()

                def compute(gb, eb, sb):
                    @pl.loop(0, EPB)
                    def _(i):
                        for j in range(DH // 16):
                            sl = pl.ds(j * 16, 16)
                            sb[i, sl] = jnp.maximum(
                                gb[i, sl] + eb[i, sl], 0.0)

                def drain(gb, eb, sb, sg, se, ss, blk, refire):
                    wait_in(gb, eb, sg, se, blk)

                    @pl.when(blk >= 2)
                    def _():
                        wait_scat(sb, ss, blk)

                    compute(gb, eb, sb)
                    pltpu.async_copy(sb, agg_sh.at[didx_r.at[blk]], ss,
                                     add=True)
                    if refire:
                        fire(gb, eb, sg, se, blk + 2)

                fire(gb0, eb0, sg0, se0, 0)
                fire(gb1, eb1, sg1, se1, 1)

                @pl.loop(0, (NBLK_R - 2) // 2)
                def _(p):
                    drain(gb0, eb0, sb0, sg0, se0, ss0, 2 * p, True)
                    drain(gb1, eb1, sb1, sg1, se1, ss1, 2 * p + 1, True)

                drain(gb0, eb0, sb0, sg0, se0, ss0, NBLK_R - 2, False)
                drain(gb1, eb1, sb1, sg1, se1, ss1, NBLK_R - 1, False)
                wait_scat(sb0, ss0, NBLK_R - 2)
                wait_scat(sb1, ss1, NBLK_R - 1)

        @pl.when(c == 0)
        def _():
            edge_loop(ha_hbm, ea_hbm)

        @pl.when(c == 1)
        def _():
            edge_loop(hb_hbm, eb_hbm)

        plsc.subcore_barrier()

        @pl.when(c == 0)
        def _():
            @pl.loop(s * RCH, N_NODES_C, step=NSUB * RCH)
            def _(r0):
                pltpu.sync_copy(agg_sh.at[pl.ds(r0, RCH)],
                                oa_hbm.at[pl.ds(r0, RCH)])

        @pl.when(c == 1)
        def _():
            @pl.loop(s * RCH, N_NODES_C, step=NSUB * RCH)
            def _(r0):
                pltpu.sync_copy(agg_sh.at[pl.ds(r0, RCH)],
                                ob_hbm.at[pl.ds(r0, RCH)])

    return sc_kernel(h0a, h0b, Ea, Eb, src, dst4, inita, initb)


def _tc_hidden_pool(h0a, h0b, agga, aggb, W2, batch3):
    B = 1000

    def body(ha_ref, hb_ref, aa_ref, ab_ref, w2_ref, b_ref,
             sums_ref, cnts_ref):
        i = pl.program_id(0)
        h0 = jnp.concatenate([ha_ref[...], hb_ref[...]], axis=1)
        agg = jnp.concatenate([aa_ref[...], ab_ref[...]], axis=1)
        h = jnp.maximum(
            h0 + jnp.dot(agg, w2_ref[...],
                         preferred_element_type=jnp.float32), 0.0)
        b = b_ref[...].reshape(B)
        onehot = (b[:, None] == lax.broadcasted_iota(
            jnp.int32, (B, N_GRAPHS_C), 1)).astype(jnp.float32)
        ps = lax.dot_general(onehot, h, (((0,), (0,)), ((), ())),
                             preferred_element_type=jnp.float32)
        pc = jnp.sum(onehot, axis=0, keepdims=True)

        @pl.when(i == 0)
        def _():
            sums_ref[...] = jnp.zeros_like(sums_ref)
            cnts_ref[...] = jnp.zeros_like(cnts_ref)

        sums_ref[...] += ps
        cnts_ref[...] += pc

    return pl.pallas_call(
        body,
        grid=(N_NODES_C // B,),
        in_specs=[
            pl.BlockSpec((B, DH), lambda i: (i, 0)),
            pl.BlockSpec((B, DH), lambda i: (i, 0)),
            pl.BlockSpec((B, DH), lambda i: (i, 0)),
            pl.BlockSpec((B, DH), lambda i: (i, 0)),
            pl.BlockSpec((HIDDEN_C, HIDDEN_C), lambda i: (0, 0)),
            pl.BlockSpec((1, 1, B), lambda i: (i, 0, 0)),
        ],
        out_specs=[
            pl.BlockSpec((N_GRAPHS_C, HIDDEN_C), lambda i: (0, 0)),
            pl.BlockSpec((1, N_GRAPHS_C), lambda i: (0, 0)),
        ],
        out_shape=[
            jax.ShapeDtypeStruct((N_GRAPHS_C, HIDDEN_C), jnp.float32),
            jax.ShapeDtypeStruct((1, N_GRAPHS_C), jnp.float32),
        ],
    )(h0a, h0b, agga, aggb, W2, batch3)


def _tc_head(sums, cnts, Wc1, bc1, Wc2, bc2):
    def body(s_ref, c_ref, w1_ref, b1_ref, w2_ref, b2_ref, o_ref):
        cnt = jnp.maximum(c_ref[...].reshape(N_GRAPHS_C), 1.0)
        g = s_ref[...] / cnt[:, None]
        z = jnp.maximum(
            jnp.dot(g, w1_ref[...], preferred_element_type=jnp.float32)
            + b1_ref[...], 0.0)
        o_ref[...] = (jnp.dot(z, w2_ref[...],
                              preferred_element_type=jnp.float32)
                      + b2_ref[...])

    return pl.pallas_call(
        body,
        out_shape=jax.ShapeDtypeStruct((N_GRAPHS_C, N_CLASSES_C), jnp.float32),
    )(sums, cnts, Wc1, bc1.reshape(1, -1), Wc2, bc2.reshape(1, -1))


def kernel(x, edge_index, edge_attr, batch, W1, We, W2, Wc1, bc1, Wc2, bc2):
    src = edge_index[0]
    dst = edge_index[1]
    h0a, h0b = _tc_h0(x, W1)

    src1, src2 = src[:CHUNK1], src[CHUNK1:]
    dst1, dst2 = dst[:CHUNK1], dst[CHUNK1:]
    nr1 = CHUNK1 // (NSUB * ROUND)
    nr2 = CHUNK2 // (NSUB * ROUND)
    Ea1, Eb1 = _tc_edge(edge_attr[:CHUNK1], We)
    Ea2, Eb2 = _tc_edge(edge_attr[CHUNK1:], We)

    zrows = jnp.zeros((N_NODES_C, DH), dtype=jnp.float32)
    p1a, p1b = _sc_aggregate(h0a, h0b, Ea1, Eb1, src1,
                             dst1.reshape(NSUB, nr1, NBLK_R, EPB),
                             zrows, zrows, nr1)
    agga, aggb = _sc_aggregate(h0a, h0b, Ea2, Eb2, src2,
                               dst2.reshape(NSUB, nr2, NBLK_R, EPB),
                               p1a, p1b, nr2)

    batch3 = batch.reshape(N_NODES_C // 1000, 1, 1000)
    sums, cnts = _tc_hidden_pool(h0a, h0b, agga, aggb, W2, batch3)
    return _tc_head(sums, cnts, Wc1, bc1, Wc2, bc2)


# speedup vs baseline: 3.8797x; 1.0277x over previous
import functools

import jax
import jax.numpy as jnp
from jax import lax
from jax.experimental import pallas as pl
from jax.experimental.pallas import tpu as pltpu
from jax.experimental.pallas import tpu_sc as plsc

N_NODES_C = 10000
N_EDGES_C = 160000
D_FEAT_C = 256
HIDDEN_C = 256
DH = 128
N_GRAPHS_C = 64
N_CLASSES_C = 10

NSUB = 16
EPB = 40
ROUND = 2000
NBLK_R = ROUND // EPB
CHUNK1 = 64000
CHUNK2 = 96000


def _tc_h0(x, W1):
    B = 2000

    def body(x_ref, w_ref, oa_ref, ob_ref):
        h = jnp.dot(x_ref[...], w_ref[...], preferred_element_type=jnp.float32)
        oa_ref[...] = h[:, :DH]
        ob_ref[...] = h[:, DH:]

    return pl.pallas_call(
        body,
        grid=(N_NODES_C // B,),
        in_specs=[
            pl.BlockSpec((B, D_FEAT_C), lambda i: (i, 0)),
            pl.BlockSpec((D_FEAT_C, HIDDEN_C), lambda i: (0, 0)),
        ],
        out_specs=[
            pl.BlockSpec((B, DH), lambda i: (i, 0)),
            pl.BlockSpec((B, DH), lambda i: (i, 0)),
        ],
        out_shape=[
            jax.ShapeDtypeStruct((N_NODES_C, DH), jnp.float32),
            jax.ShapeDtypeStruct((N_NODES_C, DH), jnp.float32),
        ],
    )(x, W1)


def _tc_edge(edge_attr, We, off, n):
    B = 4000
    noff = off // B

    def body(a_ref, w_ref, oa_ref, ob_ref):
        e = jnp.dot(a_ref[...], w_ref[...], preferred_element_type=jnp.float32)
        oa_ref[...] = e[:, :DH]
        ob_ref[...] = e[:, DH:]

    return pl.pallas_call(
        body,
        grid=(n // B,),
        in_specs=[
            pl.BlockSpec((B, 16), lambda i: (i + noff, 0)),
            pl.BlockSpec((16, HIDDEN_C), lambda i: (0, 0)),
        ],
        out_specs=[
            pl.BlockSpec((B, DH), lambda i: (i, 0)),
            pl.BlockSpec((B, DH), lambda i: (i, 0)),
        ],
        out_shape=[
            jax.ShapeDtypeStruct((n, DH), jnp.float32),
            jax.ShapeDtypeStruct((n, DH), jnp.float32),
        ],
    )(edge_attr, We)


def _sc_aggregate(h0a, h0b, Ea, Eb, src, dst4, inita, initb, nround):
    mesh = plsc.VectorSubcoreMesh(core_axis_name="c", subcore_axis_name="s")
    edges_per_sub = nround * ROUND

    @functools.partial(
        pl.kernel,
        out_type=[
            jax.ShapeDtypeStruct((N_NODES_C, DH), jnp.float32),
            jax.ShapeDtypeStruct((N_NODES_C, DH), jnp.float32),
        ],
        mesh=mesh,
        scratch_types=[
            pltpu.VMEM((ROUND,), jnp.int32),
            pltpu.VMEM((ROUND,), jnp.int32),
            pltpu.VMEM((NBLK_R, EPB), jnp.int32),
            pltpu.VMEM((NBLK_R, EPB), jnp.int32),
            pltpu.VMEM((EPB, DH), jnp.float32),
            pltpu.VMEM((EPB, DH), jnp.float32),
            pltpu.VMEM((EPB, DH), jnp.float32),
            pltpu.VMEM((EPB, DH), jnp.float32),
            pltpu.VMEM((EPB, DH), jnp.float32),
            pltpu.VMEM((EPB, DH), jnp.float32),
            pltpu.VMEM_SHARED((N_NODES_C, DH), jnp.float32),
            pltpu.SemaphoreType.DMA,
            pltpu.SemaphoreType.DMA,
            pltpu.SemaphoreType.DMA,
            pltpu.SemaphoreType.DMA,
            pltpu.SemaphoreType.DMA,
            pltpu.SemaphoreType.DMA,
            pltpu.SemaphoreType.DMA,
            pltpu.SemaphoreType.DMA,
        ],
    )
    def sc_kernel(ha_hbm, hb_hbm, ea_hbm, eb_hbm, src_hbm, dst4_hbm,
                  ia_hbm, ib_hbm, oa_hbm, ob_hbm,
                  six0, six1, dix0, dix1, gb0, gb1, eb0, eb1, sb0, sb1,
                  agg_sh, sg0, sg1, se0, se1, ss0, ss1, sem_mv, sem_ix):
        c = lax.axis_index("c")
        s = lax.axis_index("s")
        ebase = s * edges_per_sub
        RCH = 40
        sixs = [six0, six1]
        dixs = [dix0, dix1]

        def stage_idx(r):
            si, di = sixs[r % 2], dixs[r % 2]
            pltpu.async_copy(
                src_hbm.at[pl.ds(ebase + r * ROUND, ROUND)], si, sem_ix)
            pltpu.async_copy(dst4_hbm.at[s, r], di, sem_ix)

        def wait_idx(r):
            si, di = sixs[r % 2], dixs[r % 2]
            pltpu.make_async_copy(
                src_hbm.at[pl.ds(ebase + r * ROUND, ROUND)],
                si, sem_ix).wait()
            pltpu.make_async_copy(dst4_hbm.at[s, r], di, sem_ix).wait()

        stage_idx(0)

        def init_loop(i_hbm):
            @pl.loop(s * RCH, N_NODES_C, step=NSUB * RCH)
            def _(r0):
                pltpu.async_copy(i_hbm.at[pl.ds(r0, RCH)],
                                 agg_sh.at[pl.ds(r0, RCH)], sem_mv)

            @pl.loop(s * RCH, N_NODES_C, step=NSUB * RCH)
            def _(r0):
                pltpu.make_async_copy(i_hbm.at[pl.ds(r0, RCH)],
                                      agg_sh.at[pl.ds(r0, RCH)],
                                      sem_mv).wait()

        @pl.when(c == 0)
        def _():
            init_loop(ia_hbm)

        @pl.when(c == 1)
        def _():
            init_loop(ib_hbm)

        plsc.subcore_barrier()

        def edge_loop(h_hbm, e_hbm):
            for r in range(nround):
                rb = ebase + r * ROUND
                sidx_r, didx_r = sixs[r % 2], dixs[r % 2]
                wait_idx(r)
                if r + 1 < nround:
                    stage_idx(r + 1)

                def fire(gb, eb, sg, se, blk):
                    pltpu.async_copy(
                        h_hbm.at[sidx_r.at[pl.ds(blk * EPB, EPB)]], gb, sg)
                    pltpu.async_copy(
                        e_hbm.at[pl.ds(rb + blk * EPB, EPB)], eb, se)

                def wait_in(gb, eb, sg, se, blk):
                    pltpu.make_async_copy(
                        h_hbm.at[sidx_r.at[pl.ds(blk * EPB, EPB)]],
                        gb, sg).wait()
                    pltpu.make_async_copy(
                        e_hbm.at[pl.ds(rb + blk * EPB, EPB)], eb, se).wait()

                def wait_scat(sb, ss, blk):
                    pltpu.make_async_copy(
                        sb, agg_sh.at[didx_r.at[blk]], ss).wait()

                def compute(gb, eb, sb):
                    @pl.loop(0, EPB)
                    def _(i):
                        for j in range(DH // 16):
                            sl = pl.ds(j * 16, 16)
                            sb[i, sl] = jnp.maximum(
                                gb[i, sl] + eb[i, sl], 0.0)

                def drain(gb, eb, sb, sg, se, ss, blk, refire):
                    wait_in(gb, eb, sg, se, blk)

                    @pl.when(blk >= 2)
                    def _():
                        wait_scat(sb, ss, blk)

                    compute(gb, eb, sb)
                    pltpu.async_copy(sb, agg_sh.at[didx_r.at[blk]], ss,
                                     add=True)
                    if refire:
                        fire(gb, eb, sg, se, blk + 2)

                fire(gb0, eb0, sg0, se0, 0)
                fire(gb1, eb1, sg1, se1, 1)

                @pl.loop(0, (NBLK_R - 2) // 2)
                def _(p):
                    drain(gb0, eb0, sb0, sg0, se0, ss0, 2 * p, True)
                    drain(gb1, eb1, sb1, sg1, se1, ss1, 2 * p + 1, True)

                drain(gb0, eb0, sb0, sg0, se0, ss0, NBLK_R - 2, False)
                drain(gb1, eb1, sb1, sg1, se1, ss1, NBLK_R - 1, False)
                wait_scat(sb0, ss0, NBLK_R - 2)
                wait_scat(sb1, ss1, NBLK_R - 1)

        @pl.when(c == 0)
        def _():
            edge_loop(ha_hbm, ea_hbm)

        @pl.when(c == 1)
        def _():
            edge_loop(hb_hbm, eb_hbm)

        plsc.subcore_barrier()

        def out_loop(o_hbm):
            @pl.loop(s * RCH, N_NODES_C, step=NSUB * RCH)
            def _(r0):
                pltpu.async_copy(agg_sh.at[pl.ds(r0, RCH)],
                                 o_hbm.at[pl.ds(r0, RCH)], sem_mv)

            @pl.loop(s * RCH, N_NODES_C, step=NSUB * RCH)
            def _(r0):
                pltpu.make_async_copy(agg_sh.at[pl.ds(r0, RCH)],
                                      o_hbm.at[pl.ds(r0, RCH)],
                                      sem_mv).wait()

        @pl.when(c == 0)
        def _():
            out_loop(oa_hbm)

        @pl.when(c == 1)
        def _():
            out_loop(ob_hbm)

    return sc_kernel(h0a, h0b, Ea, Eb, src, dst4, inita, initb)


def _tc_hidden_pool(h0a, h0b, agga, aggb, W2, batch3):
    B = 1000

    def body(ha_ref, hb_ref, aa_ref, ab_ref, w2_ref, b_ref,
             sums_ref, cnts_ref):
        i = pl.program_id(0)
        h0 = jnp.concatenate([ha_ref[...], hb_ref[...]], axis=1)
        agg = jnp.concatenate([aa_ref[...], ab_ref[...]], axis=1)
        h = jnp.maximum(
            h0 + jnp.dot(agg, w2_ref[...],
                         preferred_element_type=jnp.float32), 0.0)
        b = b_ref[...].reshape(B)
        onehot = (b[:, None] == lax.broadcasted_iota(
            jnp.int32, (B, N_GRAPHS_C), 1)).astype(jnp.float32)
        ps = lax.dot_general(onehot, h, (((0,), (0,)), ((), ())),
                             preferred_element_type=jnp.float32)
        pc = jnp.sum(onehot, axis=0, keepdims=True)

        @pl.when(i == 0)
        def _():
            sums_ref[...] = jnp.zeros_like(sums_ref)
            cnts_ref[...] = jnp.zeros_like(cnts_ref)

        sums_ref[...] += ps
        cnts_ref[...] += pc

    return pl.pallas_call(
        body,
        grid=(N_NODES_C // B,),
        in_specs=[
            pl.BlockSpec((B, DH), lambda i: (i, 0)),
            pl.BlockSpec((B, DH), lambda i: (i, 0)),
            pl.BlockSpec((B, DH), lambda i: (i, 0)),
            pl.BlockSpec((B, DH), lambda i: (i, 0)),
            pl.BlockSpec((HIDDEN_C, HIDDEN_C), lambda i: (0, 0)),
            pl.BlockSpec((1, 1, B), lambda i: (i, 0, 0)),
        ],
        out_specs=[
            pl.BlockSpec((N_GRAPHS_C, HIDDEN_C), lambda i: (0, 0)),
            pl.BlockSpec((1, N_GRAPHS_C), lambda i: (0, 0)),
        ],
        out_shape=[
            jax.ShapeDtypeStruct((N_GRAPHS_C, HIDDEN_C), jnp.float32),
            jax.ShapeDtypeStruct((1, N_GRAPHS_C), jnp.float32),
        ],
    )(h0a, h0b, agga, aggb, W2, batch3)


def _tc_head(sums, cnts, Wc1, bc1, Wc2, bc2):
    def body(s_ref, c_ref, w1_ref, b1_ref, w2_ref, b2_ref, o_ref):
        cnt = jnp.maximum(c_ref[...].reshape(N_GRAPHS_C), 1.0)
        g = s_ref[...] / cnt[:, None]
        z = jnp.maximum(
            jnp.dot(g, w1_ref[...], preferred_element_type=jnp.float32)
            + b1_ref[...], 0.0)
        o_ref[...] = (jnp.dot(z, w2_ref[...],
                              preferred_element_type=jnp.float32)
                      + b2_ref[...])

    return pl.pallas_call(
        body,
        out_shape=jax.ShapeDtypeStruct((N_GRAPHS_C, N_CLASSES_C), jnp.float32),
    )(sums, cnts, Wc1, bc1.reshape(1, -1), Wc2, bc2.reshape(1, -1))


def kernel(x, edge_index, edge_attr, batch, W1, We, W2, Wc1, bc1, Wc2, bc2):
    src = edge_index[0]
    dst = edge_index[1]
    h0a, h0b = _tc_h0(x, W1)

    src1, src2 = src[:CHUNK1], src[CHUNK1:]
    dst1, dst2 = dst[:CHUNK1], dst[CHUNK1:]
    nr1 = CHUNK1 // (NSUB * ROUND)
    nr2 = CHUNK2 // (NSUB * ROUND)
    Ea1, Eb1 = _tc_edge(edge_attr, We, 0, CHUNK1)
    Ea2, Eb2 = _tc_edge(edge_attr, We, CHUNK1, CHUNK2)

    zrows = jnp.zeros((N_NODES_C, DH), dtype=jnp.float32)
    p1a, p1b = _sc_aggregate(h0a, h0b, Ea1, Eb1, src1,
                             dst1.reshape(NSUB, nr1, NBLK_R, EPB),
                             zrows, zrows, nr1)
    agga, aggb = _sc_aggregate(h0a, h0b, Ea2, Eb2, src2,
                               dst2.reshape(NSUB, nr2, NBLK_R, EPB),
                               p1a, p1b, nr2)

    batch3 = batch.reshape(N_NODES_C // 1000, 1, 1000)
    sums, cnts = _tc_hidden_pool(h0a, h0b, agga, aggb, W2, batch3)
    return _tc_head(sums, cnts, Wc1, bc1, Wc2, bc2)


# speedup vs baseline: 4.0475x; 1.0433x over previous
import functools

import jax
import jax.numpy as jnp
from jax import lax
from jax.experimental import pallas as pl
from jax.experimental.pallas import tpu as pltpu
from jax.experimental.pallas import tpu_sc as plsc

N_NODES_C = 10000
N_EDGES_C = 160000
D_FEAT_C = 256
HIDDEN_C = 256
DH = 128
N_GRAPHS_C = 64
N_CLASSES_C = 10

NSUB = 16
EPB = 40
ROUND = 2000
NBLK_R = ROUND // EPB
CHUNK1 = 64000
CHUNK2 = 96000


def _tc_h0(x, W1):
    B = 2000

    def body(x_ref, w_ref, oa_ref, ob_ref):
        h = jnp.dot(x_ref[...], w_ref[...], preferred_element_type=jnp.float32)
        oa_ref[...] = h[:, :DH]
        ob_ref[...] = h[:, DH:]

    return pl.pallas_call(
        body,
        grid=(N_NODES_C // B,),
        in_specs=[
            pl.BlockSpec((B, D_FEAT_C), lambda i: (i, 0)),
            pl.BlockSpec((D_FEAT_C, HIDDEN_C), lambda i: (0, 0)),
        ],
        out_specs=[
            pl.BlockSpec((B, DH), lambda i: (i, 0)),
            pl.BlockSpec((B, DH), lambda i: (i, 0)),
        ],
        out_shape=[
            jax.ShapeDtypeStruct((N_NODES_C, DH), jnp.float32),
            jax.ShapeDtypeStruct((N_NODES_C, DH), jnp.float32),
        ],
    )(x, W1)


def _tc_edge(edge_attr, We, off, n):
    B = 4000
    noff = off // B

    def body(a_ref, w_ref, oa_ref, ob_ref):
        e = jnp.dot(a_ref[...], w_ref[...], preferred_element_type=jnp.float32)
        oa_ref[...] = e[:, :DH]
        ob_ref[...] = e[:, DH:]

    return pl.pallas_call(
        body,
        grid=(n // B,),
        in_specs=[
            pl.BlockSpec((B, 16), lambda i: (i + noff, 0)),
            pl.BlockSpec((16, HIDDEN_C), lambda i: (0, 0)),
        ],
        out_specs=[
            pl.BlockSpec((B, DH), lambda i: (i, 0)),
            pl.BlockSpec((B, DH), lambda i: (i, 0)),
        ],
        out_shape=[
            jax.ShapeDtypeStruct((n, DH), jnp.float32),
            jax.ShapeDtypeStruct((n, DH), jnp.float32),
        ],
    )(edge_attr, We)


def _sc_aggregate(h0a, h0b, Ea, Eb, src, dst4, inita, initb, nround):
    mesh = plsc.VectorSubcoreMesh(core_axis_name="c", subcore_axis_name="s")
    edges_per_sub = nround * ROUND

    @functools.partial(
        pl.kernel,
        out_type=[
            jax.ShapeDtypeStruct((N_NODES_C, DH), jnp.float32),
            jax.ShapeDtypeStruct((N_NODES_C, DH), jnp.float32),
        ],
        mesh=mesh,
        scratch_types=[
            pltpu.VMEM((ROUND,), jnp.int32),
            pltpu.VMEM((ROUND,), jnp.int32),
            pltpu.VMEM((NBLK_R, EPB), jnp.int32),
            pltpu.VMEM((NBLK_R, EPB), jnp.int32),
            pltpu.VMEM((EPB, DH), jnp.float32),
            pltpu.VMEM((EPB, DH), jnp.float32),
            pltpu.VMEM((EPB, DH), jnp.float32),
            pltpu.VMEM((EPB, DH), jnp.float32),
            pltpu.VMEM((EPB, DH), jnp.float32),
            pltpu.VMEM((EPB, DH), jnp.float32),
            pltpu.VMEM_SHARED((N_NODES_C, DH), jnp.float32),
            pltpu.SemaphoreType.DMA,
            pltpu.SemaphoreType.DMA,
            pltpu.SemaphoreType.DMA,
            pltpu.SemaphoreType.DMA,
            pltpu.SemaphoreType.DMA,
            pltpu.SemaphoreType.DMA,
            pltpu.SemaphoreType.DMA,
            pltpu.SemaphoreType.DMA,
        ],
    )
    def sc_kernel(ha_hbm, hb_hbm, ea_hbm, eb_hbm, src_hbm, dst4_hbm,
                  ia_hbm, ib_hbm, oa_hbm, ob_hbm,
                  six0, six1, dix0, dix1, gb0, gb1, eb0, eb1, sb0, sb1,
                  agg_sh, sg0, sg1, se0, se1, ss0, ss1, sem_mv, sem_ix):
        c = lax.axis_index("c")
        s = lax.axis_index("s")
        ebase = s * edges_per_sub
        RCH = 40
        sixs = [six0, six1]
        dixs = [dix0, dix1]

        def stage_idx(r):
            si, di = sixs[r % 2], dixs[r % 2]
            pltpu.async_copy(
                src_hbm.at[pl.ds(ebase + r * ROUND, ROUND)], si, sem_ix)
            pltpu.async_copy(dst4_hbm.at[s, r], di, sem_ix)

        def wait_idx(r):
            si, di = sixs[r % 2], dixs[r % 2]
            pltpu.make_async_copy(
                src_hbm.at[pl.ds(ebase + r * ROUND, ROUND)],
                si, sem_ix).wait()
            pltpu.make_async_copy(dst4_hbm.at[s, r], di, sem_ix).wait()

        stage_idx(0)

        def init_loop(i_hbm):
            @pl.loop(s * RCH, N_NODES_C, step=NSUB * RCH)
            def _(r0):
                pltpu.async_copy(i_hbm.at[pl.ds(r0, RCH)],
                                 agg_sh.at[pl.ds(r0, RCH)], sem_mv)

            @pl.loop(s * RCH, N_NODES_C, step=NSUB * RCH)
            def _(r0):
                pltpu.make_async_copy(i_hbm.at[pl.ds(r0, RCH)],
                                      agg_sh.at[pl.ds(r0, RCH)],
                                      sem_mv).wait()

        @pl.when(c == 0)
        def _():
            init_loop(ia_hbm)

        @pl.when(c == 1)
        def _():
            init_loop(ib_hbm)

        plsc.subcore_barrier()

        def edge_loop(h_hbm, e_hbm):
            for r in range(nround):
                rb = ebase + r * ROUND
                sidx_r, didx_r = sixs[r % 2], dixs[r % 2]
                wait_idx(r)
                if r + 1 < nround:
                    stage_idx(r + 1)

                def fire(gb, eb, sg, se, blk):
                    pltpu.async_copy(
                        h_hbm.at[sidx_r.at[pl.ds(blk * EPB, EPB)]], gb, sg)
                    pltpu.async_copy(
                        e_hbm.at[pl.ds(rb + blk * EPB, EPB)], eb, se)

                def wait_in(gb, eb, sg, se, blk):
                    pltpu.make_async_copy(
                        h_hbm.at[sidx_r.at[pl.ds(blk * EPB, EPB)]],
                        gb, sg).wait()
                    pltpu.make_async_copy(
                        e_hbm.at[pl.ds(rb + blk * EPB, EPB)], eb, se).wait()

                def wait_scat(sb, ss, blk):
                    pltpu.make_async_copy(
                        sb, agg_sh.at[didx_r.at[blk]], ss).wait()

                def compute(gb, eb, sb):
                    @pl.loop(0, EPB)
                    def _(i):
                        for j in range(DH // 16):
                            sl = pl.ds(j * 16, 16)
                            sb[i, sl] = jnp.maximum(
                                gb[i, sl] + eb[i, sl], 0.0)

                def drain(gb, eb, sb, sg, se, ss, blk, refire):
                    wait_in(gb, eb, sg, se, blk)

                    @pl.when(blk >= 2)
                    def _():
                        wait_scat(sb, ss, blk)

                    compute(gb, eb, sb)
                    pltpu.async_copy(sb, agg_sh.at[didx_r.at[blk]], ss,
                                     add=True)
                    if refire:
                        fire(gb, eb, sg, se, blk + 2)

                fire(gb0, eb0, sg0, se0, 0)
                fire(gb1, eb1, sg1, se1, 1)

                @pl.loop(0, (NBLK_R - 2) // 2)
                def _(p):
                    drain(gb0, eb0, sb0, sg0, se0, ss0, 2 * p, True)
                    drain(gb1, eb1, sb1, sg1, se1, ss1, 2 * p + 1, True)

                drain(gb0, eb0, sb0, sg0, se0, ss0, NBLK_R - 2, False)
                drain(gb1, eb1, sb1, sg1, se1, ss1, NBLK_R - 1, False)
                wait_scat(sb0, ss0, NBLK_R - 2)
                wait_scat(sb1, ss1, NBLK_R - 1)

        @pl.when(c == 0)
        def _():
            edge_loop(ha_hbm, ea_hbm)

        @pl.when(c == 1)
        def _():
            edge_loop(hb_hbm, eb_hbm)

        plsc.subcore_barrier()

        def out_loop(o_hbm):
            @pl.loop(s * RCH, N_NODES_C, step=NSUB * RCH)
            def _(r0):
                pltpu.async_copy(agg_sh.at[pl.ds(r0, RCH)],
                                 o_hbm.at[pl.ds(r0, RCH)], sem_mv)

            @pl.loop(s * RCH, N_NODES_C, step=NSUB * RCH)
            def _(r0):
                pltpu.make_async_copy(agg_sh.at[pl.ds(r0, RCH)],
                                      o_hbm.at[pl.ds(r0, RCH)],
                                      sem_mv).wait()

        @pl.when(c == 0)
        def _():
            out_loop(oa_hbm)

        @pl.when(c == 1)
        def _():
            out_loop(ob_hbm)

    return sc_kernel(h0a, h0b, Ea, Eb, src, dst4, inita, initb)


def _tc_hidden_pool(h0a, h0b, agga, aggb, W2, batch3):
    B = 1000

    def body(ha_ref, hb_ref, aa_ref, ab_ref, w2_ref, b_ref,
             sums_ref, cnts_ref):
        i = pl.program_id(0)
        h0 = jnp.concatenate([ha_ref[...], hb_ref[...]], axis=1)
        agg = jnp.concatenate([aa_ref[...], ab_ref[...]], axis=1)
        h = jnp.maximum(
            h0 + jnp.dot(agg, w2_ref[...],
                         preferred_element_type=jnp.float32), 0.0)
        b = b_ref[...].reshape(B)
        onehot = (b[:, None] == lax.broadcasted_iota(
            jnp.int32, (B, N_GRAPHS_C), 1)).astype(jnp.float32)
        ps = lax.dot_general(onehot, h, (((0,), (0,)), ((), ())),
                             preferred_element_type=jnp.float32)
        pc = jnp.sum(onehot, axis=0, keepdims=True)

        @pl.when(i == 0)
        def _():
            sums_ref[...] = jnp.zeros_like(sums_ref)
            cnts_ref[...] = jnp.zeros_like(cnts_ref)

        sums_ref[...] += ps
        cnts_ref[...] += pc

    return pl.pallas_call(
        body,
        grid=(N_NODES_C // B,),
        in_specs=[
            pl.BlockSpec((B, DH), lambda i: (i, 0)),
            pl.BlockSpec((B, DH), lambda i: (i, 0)),
            pl.BlockSpec((B, DH), lambda i: (i, 0)),
            pl.BlockSpec((B, DH), lambda i: (i, 0)),
            pl.BlockSpec((HIDDEN_C, HIDDEN_C), lambda i: (0, 0)),
            pl.BlockSpec((1, 1, B), lambda i: (i, 0, 0)),
        ],
        out_specs=[
            pl.BlockSpec((N_GRAPHS_C, HIDDEN_C), lambda i: (0, 0)),
            pl.BlockSpec((1, N_GRAPHS_C), lambda i: (0, 0)),
        ],
        out_shape=[
            jax.ShapeDtypeStruct((N_GRAPHS_C, HIDDEN_C), jnp.float32),
            jax.ShapeDtypeStruct((1, N_GRAPHS_C), jnp.float32),
        ],
    )(h0a, h0b, agga, aggb, W2, batch3)


def _tc_head(sums, cnts, Wc1, bc1, Wc2, bc2):
    def body(s_ref, c_ref, w1_ref, b1_ref, w2_ref, b2_ref, o_ref):
        cnt = jnp.maximum(c_ref[...].reshape(N_GRAPHS_C), 1.0)
        g = s_ref[...] / cnt[:, None]
        z = jnp.maximum(
            jnp.dot(g, w1_ref[...], preferred_element_type=jnp.float32)
            + b1_ref[...], 0.0)
        o_ref[...] = (jnp.dot(z, w2_ref[...],
                              preferred_element_type=jnp.float32)
                      + b2_ref[...])

    return pl.pallas_call(
        body,
        out_shape=jax.ShapeDtypeStruct((N_GRAPHS_C, N_CLASSES_C), jnp.float32),
    )(sums, cnts, Wc1, bc1.reshape(1, -1), Wc2, bc2.reshape(1, -1))


def kernel(x, edge_index, edge_attr, batch, W1, We, W2, Wc1, bc1, Wc2, bc2):
    src = edge_index[0]
    dst = edge_index[1]
    h0a, h0b = _tc_h0(x, W1)

    src1, src2 = src[:CHUNK1], src[CHUNK1:]
    dst1, dst2 = dst[:CHUNK1], dst[CHUNK1:]
    nr1 = CHUNK1 // (NSUB * ROUND)
    nr2 = CHUNK2 // (NSUB * ROUND)
    Ea1, Eb1 = _tc_edge(edge_attr[:CHUNK1], We, 0, CHUNK1)
    Ea2, Eb2 = _tc_edge(edge_attr[CHUNK1:], We, 0, CHUNK2)

    zrows = jnp.zeros((N_NODES_C, DH), dtype=jnp.float32)
    p1a, p1b = _sc_aggregate(h0a, h0b, Ea1, Eb1, src1,
                             dst1.reshape(NSUB, nr1, NBLK_R, EPB),
                             zrows, zrows, nr1)
    agga, aggb = _sc_aggregate(h0a, h0b, Ea2, Eb2, src2,
                               dst2.reshape(NSUB, nr2, NBLK_R, EPB),
                               p1a, p1b, nr2)

    batch3 = batch.reshape(N_NODES_C // 1000, 1, 1000)
    sums, cnts = _tc_hidden_pool(h0a, h0b, agga, aggb, W2, batch3)
    return _tc_head(sums, cnts, Wc1, bc1, Wc2, bc2)


# speedup vs baseline: 4.1948x; 1.0364x over previous
import functools

import jax
import jax.numpy as jnp
from jax import lax
from jax.experimental import pallas as pl
from jax.experimental.pallas import tpu as pltpu
from jax.experimental.pallas import tpu_sc as plsc

N_NODES_C = 10000
N_EDGES_C = 160000
D_FEAT_C = 256
HIDDEN_C = 256
DH = 128
N_GRAPHS_C = 64
N_CLASSES_C = 10

NSUB = 16
EPB = 40
ROUND = 2000
NBLK_R = ROUND // EPB
CHUNK1 = 64000
CHUNK2 = 96000


def _tc_h0(x, W1):
    B = 2000

    def body(x_ref, w_ref, oa_ref, ob_ref):
        h = jnp.dot(x_ref[...], w_ref[...], preferred_element_type=jnp.float32)
        oa_ref[...] = h[:, :DH]
        ob_ref[...] = h[:, DH:]

    return pl.pallas_call(
        body,
        grid=(N_NODES_C // B,),
        in_specs=[
            pl.BlockSpec((B, D_FEAT_C), lambda i: (i, 0)),
            pl.BlockSpec((D_FEAT_C, HIDDEN_C), lambda i: (0, 0)),
        ],
        out_specs=[
            pl.BlockSpec((B, DH), lambda i: (i, 0)),
            pl.BlockSpec((B, DH), lambda i: (i, 0)),
        ],
        out_shape=[
            jax.ShapeDtypeStruct((N_NODES_C, DH), jnp.float32),
            jax.ShapeDtypeStruct((N_NODES_C, DH), jnp.float32),
        ],
    )(x, W1)


def _tc_edge(edge_attr, We, off, n):
    B = 4000
    noff = off // B

    def body(a_ref, w_ref, oa_ref, ob_ref):
        e = jnp.dot(a_ref[...], w_ref[...], preferred_element_type=jnp.float32)
        oa_ref[...] = e[:, :DH]
        ob_ref[...] = e[:, DH:]

    return pl.pallas_call(
        body,
        grid=(n // B,),
        in_specs=[
            pl.BlockSpec((B, 16), lambda i: (i + noff, 0)),
            pl.BlockSpec((16, HIDDEN_C), lambda i: (0, 0)),
        ],
        out_specs=[
            pl.BlockSpec((B, DH), lambda i: (i, 0)),
            pl.BlockSpec((B, DH), lambda i: (i, 0)),
        ],
        out_shape=[
            jax.ShapeDtypeStruct((n, DH), jnp.float32),
            jax.ShapeDtypeStruct((n, DH), jnp.float32),
        ],
    )(edge_attr, We)


def _sc_aggregate(h0a, h0b, Ea, Eb, src, dst4, inita, initb, nround):
    mesh = plsc.VectorSubcoreMesh(core_axis_name="c", subcore_axis_name="s")
    edges_per_sub = nround * ROUND

    @functools.partial(
        pl.kernel,
        out_type=[
            jax.ShapeDtypeStruct((N_NODES_C, DH), jnp.float32),
            jax.ShapeDtypeStruct((N_NODES_C, DH), jnp.float32),
        ],
        mesh=mesh,
        scratch_types=[
            pltpu.VMEM((ROUND,), jnp.int32),
            pltpu.VMEM((ROUND,), jnp.int32),
            pltpu.VMEM((NBLK_R, EPB), jnp.int32),
            pltpu.VMEM((NBLK_R, EPB), jnp.int32),
            pltpu.VMEM((EPB, DH), jnp.float32),
            pltpu.VMEM((EPB, DH), jnp.float32),
            pltpu.VMEM((EPB, DH), jnp.float32),
            pltpu.VMEM((EPB, DH), jnp.float32),
            pltpu.VMEM((EPB, DH), jnp.float32),
            pltpu.VMEM((EPB, DH), jnp.float32),
            pltpu.VMEM_SHARED((N_NODES_C, DH), jnp.float32),
            pltpu.SemaphoreType.DMA,
            pltpu.SemaphoreType.DMA,
            pltpu.SemaphoreType.DMA,
            pltpu.SemaphoreType.DMA,
            pltpu.SemaphoreType.DMA,
            pltpu.SemaphoreType.DMA,
            pltpu.SemaphoreType.DMA,
            pltpu.SemaphoreType.DMA,
        ],
    )
    def sc_kernel(ha_hbm, hb_hbm, ea_hbm, eb_hbm, src_hbm, dst4_hbm,
                  ia_hbm, ib_hbm, oa_hbm, ob_hbm,
                  six0, six1, dix0, dix1, gb0, gb1, eb0, eb1, sb0, sb1,
                  agg_sh, sg0, sg1, se0, se1, ss0, ss1, sem_mv, sem_ix):
        c = lax.axis_index("c")
        s = lax.axis_index("s")
        ebase = s * edges_per_sub
        RCH = 40
        sixs = [six0, six1]
        dixs = [dix0, dix1]

        def stage_idx(r):
            si, di = sixs[r % 2], dixs[r % 2]
            pltpu.async_copy(
                src_hbm.at[pl.ds(ebase + r * ROUND, ROUND)], si, sem_ix)
            pltpu.async_copy(dst4_hbm.at[s, r], di, sem_ix)

        def wait_idx(r):
            si, di = sixs[r % 2], dixs[r % 2]
            pltpu.make_async_copy(
                src_hbm.at[pl.ds(ebase + r * ROUND, ROUND)],
                si, sem_ix).wait()
            pltpu.make_async_copy(dst4_hbm.at[s, r], di, sem_ix).wait()

        stage_idx(0)

        def init_loop(i_hbm):
            @pl.loop(s * RCH, N_NODES_C, step=NSUB * RCH)
            def _(r0):
                pltpu.async_copy(i_hbm.at[pl.ds(r0, RCH)],
                                 agg_sh.at[pl.ds(r0, RCH)], sem_mv)

            @pl.loop(s * RCH, N_NODES_C, step=NSUB * RCH)
            def _(r0):
                pltpu.make_async_copy(i_hbm.at[pl.ds(r0, RCH)],
                                      agg_sh.at[pl.ds(r0, RCH)],
                                      sem_mv).wait()

        @pl.when(c == 0)
        def _():
            init_loop(ia_hbm)

        @pl.when(c == 1)
        def _():
            init_loop(ib_hbm)

        plsc.subcore_barrier()

        def edge_loop(h_hbm, e_hbm):
            for r in range(nround):
                rb = ebase + r * ROUND
                sidx_r, didx_r = sixs[r % 2], dixs[r % 2]
                wait_idx(r)
                if r + 1 < nround:
                    stage_idx(r + 1)

                def fire(gb, eb, sg, se, blk):
                    pltpu.async_copy(
                        h_hbm.at[sidx_r.at[pl.ds(blk * EPB, EPB)]], gb, sg)
                    pltpu.async_copy(
                        e_hbm.at[pl.ds(rb + blk * EPB, EPB)], eb, se)

                def wait_in(gb, eb, sg, se, blk):
                    pltpu.make_async_copy(
                        h_hbm.at[sidx_r.at[pl.ds(blk * EPB, EPB)]],
                        gb, sg).wait()
                    pltpu.make_async_copy(
                        e_hbm.at[pl.ds(rb + blk * EPB, EPB)], eb, se).wait()

                def wait_scat(sb, ss, blk):
                    pltpu.make_async_copy(
                        sb, agg_sh.at[didx_r.at[blk]], ss).wait()

                def compute(gb, eb, sb):
                    @functools.partial(plsc.parallel_loop, 0, EPB,
                                       unroll=4)
                    def _(i):
                        for j in range(DH // 16):
                            sl = pl.ds(j * 16, 16)
                            sb[i, sl] = jnp.maximum(
                                gb[i, sl] + eb[i, sl], 0.0)

                def drain(gb, eb, sb, sg, se, ss, blk, refire):
                    wait_in(gb, eb, sg, se, blk)

                    @pl.when(blk >= 2)
                    def _():
                        wait_scat(sb, ss, blk)

                    compute(gb, eb, sb)
                    pltpu.async_copy(sb, agg_sh.at[didx_r.at[blk]], ss,
                                     add=True)
                    if refire:
                        fire(gb, eb, sg, se, blk + 2)

                fire(gb0, eb0, sg0, se0, 0)
                fire(gb1, eb1, sg1, se1, 1)

                @pl.loop(0, (NBLK_R - 2) // 2)
                def _(p):
                    drain(gb0, eb0, sb0, sg0, se0, ss0, 2 * p, True)
                    drain(gb1, eb1, sb1, sg1, se1, ss1, 2 * p + 1, True)

                drain(gb0, eb0, sb0, sg0, se0, ss0, NBLK_R - 2, False)
                drain(gb1, eb1, sb1, sg1, se1, ss1, NBLK_R - 1, False)
                wait_scat(sb0, ss0, NBLK_R - 2)
                wait_scat(sb1, ss1, NBLK_R - 1)

        @pl.when(c == 0)
        def _():
            edge_loop(ha_hbm, ea_hbm)

        @pl.when(c == 1)
        def _():
            edge_loop(hb_hbm, eb_hbm)

        plsc.subcore_barrier()

        def out_loop(o_hbm):
            @pl.loop(s * RCH, N_NODES_C, step=NSUB * RCH)
            def _(r0):
                pltpu.async_copy(agg_sh.at[pl.ds(r0, RCH)],
                                 o_hbm.at[pl.ds(r0, RCH)], sem_mv)

            @pl.loop(s * RCH, N_NODES_C, step=NSUB * RCH)
            def _(r0):
                pltpu.make_async_copy(agg_sh.at[pl.ds(r0, RCH)],
                                      o_hbm.at[pl.ds(r0, RCH)],
                                      sem_mv).wait()

        @pl.when(c == 0)
        def _():
            out_loop(oa_hbm)

        @pl.when(c == 1)
        def _():
            out_loop(ob_hbm)

    return sc_kernel(h0a, h0b, Ea, Eb, src, dst4, inita, initb)


def _tc_hidden_pool(h0a, h0b, agga, aggb, W2, batch3, Wc1, bc1, Wc2, bc2):
    B = 1000

    def body(ha_ref, hb_ref, aa_ref, ab_ref, w2_ref, b_ref,
             w1c_ref, b1c_ref, w2c_ref, b2c_ref,
             sums_ref, cnts_ref, o_ref):
        i = pl.program_id(0)
        h0 = jnp.concatenate([ha_ref[...], hb_ref[...]], axis=1)
        agg = jnp.concatenate([aa_ref[...], ab_ref[...]], axis=1)
        h = jnp.maximum(
            h0 + jnp.dot(agg, w2_ref[...],
                         preferred_element_type=jnp.float32), 0.0)
        b = b_ref[...].reshape(B)
        onehot = (b[:, None] == lax.broadcasted_iota(
            jnp.int32, (B, N_GRAPHS_C), 1)).astype(jnp.float32)
        ps = lax.dot_general(onehot, h, (((0,), (0,)), ((), ())),
                             preferred_element_type=jnp.float32)
        pc = jnp.sum(onehot, axis=0, keepdims=True)

        @pl.when(i == 0)
        def _():
            sums_ref[...] = jnp.zeros_like(sums_ref)
            cnts_ref[...] = jnp.zeros_like(cnts_ref)

        sums_ref[...] += ps
        cnts_ref[...] += pc

        @pl.when(i == N_NODES_C // B - 1)
        def _():
            cnt = jnp.maximum(cnts_ref[...].reshape(N_GRAPHS_C), 1.0)
            g = sums_ref[...] / cnt[:, None]
            z = jnp.maximum(
                jnp.dot(g, w1c_ref[...], preferred_element_type=jnp.float32)
                + b1c_ref[...], 0.0)
            o_ref[...] = (jnp.dot(z, w2c_ref[...],
                                  preferred_element_type=jnp.float32)
                          + b2c_ref[...])

    return pl.pallas_call(
        body,
        grid=(N_NODES_C // B,),
        in_specs=[
            pl.BlockSpec((B, DH), lambda i: (i, 0)),
            pl.BlockSpec((B, DH), lambda i: (i, 0)),
            pl.BlockSpec((B, DH), lambda i: (i, 0)),
            pl.BlockSpec((B, DH), lambda i: (i, 0)),
            pl.BlockSpec((HIDDEN_C, HIDDEN_C), lambda i: (0, 0)),
            pl.BlockSpec((1, 1, B), lambda i: (i, 0, 0)),
            pl.BlockSpec((HIDDEN_C, HIDDEN_C // 2), lambda i: (0, 0)),
            pl.BlockSpec((1, HIDDEN_C // 2), lambda i: (0, 0)),
            pl.BlockSpec((HIDDEN_C // 2, N_CLASSES_C), lambda i: (0, 0)),
            pl.BlockSpec((1, N_CLASSES_C), lambda i: (0, 0)),
        ],
        out_specs=[
            pl.BlockSpec((N_GRAPHS_C, HIDDEN_C), lambda i: (0, 0)),
            pl.BlockSpec((1, N_GRAPHS_C), lambda i: (0, 0)),
            pl.BlockSpec((N_GRAPHS_C, N_CLASSES_C), lambda i: (0, 0)),
        ],
        out_shape=[
            jax.ShapeDtypeStruct((N_GRAPHS_C, HIDDEN_C), jnp.float32),
            jax.ShapeDtypeStruct((1, N_GRAPHS_C), jnp.float32),
            jax.ShapeDtypeStruct((N_GRAPHS_C, N_CLASSES_C), jnp.float32),
        ],
    )(h0a, h0b, agga, aggb, W2, batch3,
      Wc1, bc1.reshape(1, -1), Wc2, bc2.reshape(1, -1))


def kernel(x, edge_index, edge_attr, batch, W1, We, W2, Wc1, bc1, Wc2, bc2):
    src = edge_index[0]
    dst = edge_index[1]
    h0a, h0b = _tc_h0(x, W1)

    src1, src2 = src[:CHUNK1], src[CHUNK1:]
    dst1, dst2 = dst[:CHUNK1], dst[CHUNK1:]
    nr1 = CHUNK1 // (NSUB * ROUND)
    nr2 = CHUNK2 // (NSUB * ROUND)
    Ea1, Eb1 = _tc_edge(edge_attr[:CHUNK1], We, 0, CHUNK1)
    Ea2, Eb2 = _tc_edge(edge_attr[CHUNK1:], We, 0, CHUNK2)

    zrows = jnp.zeros((N_NODES_C, DH), dtype=jnp.float32)
    p1a, p1b = _sc_aggregate(h0a, h0b, Ea1, Eb1, src1,
                             dst1.reshape(NSUB, nr1, NBLK_R, EPB),
                             zrows, zrows, nr1)
    agga, aggb = _sc_aggregate(h0a, h0b, Ea2, Eb2, src2,
                               dst2.reshape(NSUB, nr2, NBLK_R, EPB),
                               p1a, p1b, nr2)

    batch3 = batch.reshape(N_NODES_C // 1000, 1, 1000)
    _, _, out = _tc_hidden_pool(h0a, h0b, agga, aggb, W2, batch3,
                                Wc1, bc1, Wc2, bc2)
    return out
